# baseline (speedup 1.0000x reference)
import functools

import jax
import jax.numpy as jnp
from jax import lax
from jax.experimental import pallas as pl
from jax.experimental.pallas import tpu as pltpu
from jax.experimental.pallas import tpu_sc as plsc

VOCAB = 1_000_000
DIM = 64
BATCH = 16384

NC = 2
NS = 16
NW = NC * NS
BPW = BATCH // NW
CH = 128
NCH = BPW // CH
L = 16


def _permute(x, idx):
    dnums = lax.GatherDimensionNumbers(
        offset_dims=(), collapsed_slice_dims=(0,), start_index_map=(0,)
    )
    return lax.gather(
        x, idx[:, None], dnums, (1,),
        mode=lax.GatherScatterMode.PROMISE_IN_BOUNDS,
    )


def _compute_chunk(a_ref, b_ref, out_ref, base):
    lanes = lax.iota(jnp.int32, L)
    rots = [(lanes + sh) % L for sh in (8, 4, 2, 1)]

    def group_body(g, carry):
        r0 = g * L
        acc = jnp.zeros((L,), jnp.float32)
        for k in range(L):
            r = r0 + k
            s = (
                a_ref[r, pl.ds(0, L)] * b_ref[r, pl.ds(0, L)]
                + a_ref[r, pl.ds(L, L)] * b_ref[r, pl.ds(L, L)]
                + a_ref[r, pl.ds(2 * L, L)] * b_ref[r, pl.ds(2 * L, L)]
                + a_ref[r, pl.ds(3 * L, L)] * b_ref[r, pl.ds(3 * L, L)]
            )
            for rot in rots:
                s = s + _permute(s, rot)
            acc = jnp.where(lanes == k, s, acc)
        out_ref[pl.ds(base + r0, L)] = acc
        return carry

    lax.fori_loop(0, CH // L, group_body, 0)


@functools.partial(
    pl.kernel,
    out_type=jax.ShapeDtypeStruct((NW, BPW), jnp.float32),
    mesh=plsc.VectorSubcoreMesh(core_axis_name="c", subcore_axis_name="s"),
    compiler_params=pltpu.CompilerParams(use_tc_tiling_on_sc=False),
    scratch_types=[
        pltpu.VMEM((NCH, CH), jnp.int32),
        pltpu.VMEM((NCH, CH), jnp.int32),
        pltpu.VMEM((CH, DIM), jnp.float32),
        pltpu.VMEM((CH, DIM), jnp.float32),
        pltpu.VMEM((CH, DIM), jnp.float32),
        pltpu.VMEM((CH, DIM), jnp.float32),
        pltpu.VMEM((BPW,), jnp.float32),
        pltpu.SemaphoreType.DMA,
        pltpu.SemaphoreType.DMA,
    ],
)
def _w2v_sc(idx_c_hbm, idx_x_hbm, a_tab, b_tab, out_hbm,
            idx_c_v, idx_x_v, a0, a1, b0, b1, out_v, sem0, sem1):
    wid = lax.axis_index("s") * NC + lax.axis_index("c")

    pltpu.sync_copy(idx_c_hbm.at[wid], idx_c_v)
    pltpu.sync_copy(idx_x_hbm.at[wid], idx_x_v)

    a_bufs = (a0, a1)
    b_bufs = (b0, b1)
    sems = (sem0, sem1)

    def start(c):
        slot = c % 2
        ca = pltpu.async_copy(a_tab.at[idx_c_v.at[c]], a_bufs[slot], sems[slot])
        cb = pltpu.async_copy(b_tab.at[idx_x_v.at[c]], b_bufs[slot], sems[slot])
        return ca, cb

    pending = start(0)
    for c in range(NCH):
        nxt = start(c + 1) if c + 1 < NCH else None
        for cp in pending:
            cp.wait()
        slot = c % 2
        _compute_chunk(a_bufs[slot], b_bufs[slot], out_v, c * CH)
        pending = nxt

    pltpu.sync_copy(out_v, out_hbm.at[wid])


def kernel(center_words, context_words, input_embed, output_embed):
    idx_c = center_words.reshape(NW, NCH, CH)
    idx_x = context_words.reshape(NW, NCH, CH)
    out = _w2v_sc(idx_c, idx_x, input_embed, output_embed)
    return out.reshape(BATCH)


# speedup vs baseline: 1.9693x; 1.9693x over previous
import functools

import jax
import jax.numpy as jnp
from jax import lax
from jax.experimental import pallas as pl
from jax.experimental.pallas import tpu as pltpu
from jax.experimental.pallas import tpu_sc as plsc

VOCAB = 1_000_000
DIM = 64
BATCH = 16384

NC = 2
NS = 16
NW = NC * NS
BPW = BATCH // NW
L = 16

CW = 256
WIN = 62464
NCH_STD = WIN // CW
WIN15 = VOCAB - 15 * WIN
NCH15 = (WIN15 // CW // 2) * 2
TAIL_LO = 15 * WIN + NCH15 * CW
TAIL_N = VOCAB - TAIL_LO
NIDXV = BATCH // L


def _popcount(mask):
    return plsc.all_reduce_population_count(mask)[0]


def _permute(x, idx):
    dnums = lax.GatherDimensionNumbers(
        offset_dims=(), collapsed_slice_dims=(0,), start_index_map=(0,)
    )
    return lax.gather(
        x, idx[:, None], dnums, (1,),
        mode=lax.GatherScatterMode.PROMISE_IN_BOUNDS,
    )


@functools.partial(
    pl.kernel,
    out_type=(
        jax.ShapeDtypeStruct((BATCH * DIM,), jnp.float32),
        jax.ShapeDtypeStruct((BATCH * DIM,), jnp.float32),
    ),
    mesh=plsc.VectorSubcoreMesh(core_axis_name="c", subcore_axis_name="s"),
    compiler_params=pltpu.CompilerParams(needs_layout_passes=False),
    scratch_types=[
        pltpu.VMEM((BATCH,), jnp.int32),
        pltpu.VMEM((BATCH + L,), jnp.int32),
        pltpu.VMEM((BATCH + L,), jnp.int32),
        pltpu.VMEM((DIM, CW), jnp.float32),
        pltpu.VMEM((DIM, CW), jnp.float32),
        pltpu.VMEM((DIM, TAIL_N), jnp.float32),
        pltpu.VMEM((L, DIM), jnp.float32),
        pltpu.SemaphoreType.DMA,
        pltpu.SemaphoreType.DMA,
        pltpu.SemaphoreType.DMA,
    ],
)
def _w2v_extract(idx_c_hbm, idx_x_hbm, ta, tb, out_a, out_b,
                 idxv, bkt_w, bkt_p, ch0, ch1, chtail,
                 rowbufs, sem0, sem1, wsem):
    core = lax.axis_index("c")
    tid = lax.axis_index("s")
    lanes = lax.iota(jnp.int32, L)
    bufs = (ch0, ch1)
    sems = (sem0, sem1)

    lo = tid * WIN
    hi = jnp.where(tid == NS - 1, VOCAB, lo + WIN)
    nch = jnp.where(tid == NS - 1, NCH15, NCH_STD)

    def pipe(tab, idx_hbm, out_hbm):
        pltpu.sync_copy(idx_hbm, idxv)

        def scan_body(v, cnt):
            w = idxv[pl.ds(v * L, L)]
            pos = v * L + lanes
            m = (w >= lo) & (w < hi)
            sk, sv, _ = plsc.sort_key_val(w, pos, mask=m)
            bkt_w[pl.ds(cnt, L)] = sk
            bkt_p[pl.ds(cnt, L)] = sv
            return cnt + _popcount(m)

        cnt = lax.fori_loop(0, NIDXV, scan_body, 0)
        nbv = (cnt + L - 1) >> 4

        def extract(chunk_lo, buf, cw):
            def bv_body(v, carry):
                w = bkt_w[pl.ds(v * L, L)]
                p = bkt_p[pl.ds(v * L, L)]
                m = (w >= chunk_lo) & (w < chunk_lo + cw)
                cm = _popcount(m)
                wv_abs, pv, _ = plsc.sort_key_val(w, p, mask=m)
                wv = wv_abs - chunk_lo

                def e_body(e, carry2):
                    ev = jnp.full((L,), e, jnp.int32)
                    col = _permute(wv, ev)
                    ps = _permute(pv, ev)[0]
                    for g in range(DIM // L):
                        vals = plsc.load_gather(buf, [lanes + g * L, col])
                        rowbufs[e, pl.ds(g * L, L)] = vals
                    off = pl.multiple_of(ps * DIM, 8)
                    pltpu.async_copy(
                        rowbufs.at[e], out_hbm.at[pl.ds(off, DIM)], wsem)
                    return carry2

                lax.fori_loop(0, cm, e_body, 0)

                def drain_body(e, carry2):
                    pltpu.make_async_copy(
                        rowbufs.at[0], out_hbm.at[pl.ds(0, DIM)], wsem
                    ).wait()
                    return carry2

                lax.fori_loop(0, cm, drain_body, 0)
                return carry

            lax.fori_loop(0, nbv, bv_body, 0)

        def issue(c, slot):
            off = pl.multiple_of(lo + c * CW, 128)
            pltpu.async_copy(tab.at[:, pl.ds(off, CW)], bufs[slot], sems[slot])

        def wait(slot):
            pltpu.make_async_copy(
                tab.at[:, pl.ds(0, CW)], bufs[slot], sems[slot]).wait()

        issue(0, 0)
        issue(1, 1)

        def pair_body(c2, carry):
            c = 2 * c2
            wait(0)
            extract(lo + c * CW, bufs[0], CW)

            @pl.when(c + 2 < nch)
            def _():
                issue(c + 2, 0)

            wait(1)
            extract(lo + (c + 1) * CW, bufs[1], CW)

            @pl.when(c + 3 < nch)
            def _():
                issue(c + 3, 1)

            return carry

        lax.fori_loop(0, nch >> 1, pair_body, 0)

        @pl.when(tid == NS - 1)
        def _():
            pltpu.sync_copy(tab.at[:, pl.ds(TAIL_LO, TAIL_N)], chtail)
            extract(TAIL_LO, chtail, TAIL_N)

    @pl.when(core == 0)
    def _():
        pipe(ta, idx_c_hbm, out_a)

    @pl.when(core == 1)
    def _():
        pipe(tb, idx_x_hbm, out_b)


CH2 = 128
NCH2 = BPW // CH2
CB = CH2 * DIM


def _dot_chunk(a_ref, b_ref, out_ref, base):
    lanes = lax.iota(jnp.int32, L)
    rots = [(lanes + sh) % L for sh in (8, 4, 2, 1)]

    def group_body(g, carry):
        r0 = g * L
        acc = jnp.zeros((L,), jnp.float32)
        for k in range(L):
            f = (r0 + k) * DIM
            s = (
                a_ref[pl.ds(f, L)] * b_ref[pl.ds(f, L)]
                + a_ref[pl.ds(f + L, L)] * b_ref[pl.ds(f + L, L)]
                + a_ref[pl.ds(f + 2 * L, L)] * b_ref[pl.ds(f + 2 * L, L)]
                + a_ref[pl.ds(f + 3 * L, L)] * b_ref[pl.ds(f + 3 * L, L)]
            )
            for rot in rots:
                s = s + _permute(s, rot)
            acc = jnp.where(lanes == k, s, acc)
        out_ref[pl.ds(base + r0, L)] = acc
        return carry

    lax.fori_loop(0, CH2 // L, group_body, 0)


@functools.partial(
    pl.kernel,
    out_type=jax.ShapeDtypeStruct((NW, BPW), jnp.float32),
    mesh=plsc.VectorSubcoreMesh(core_axis_name="c", subcore_axis_name="s"),
    scratch_types=[
        pltpu.VMEM((CB,), jnp.float32),
        pltpu.VMEM((CB,), jnp.float32),
        pltpu.VMEM((CB,), jnp.float32),
        pltpu.VMEM((CB,), jnp.float32),
        pltpu.VMEM((BPW,), jnp.float32),
        pltpu.SemaphoreType.DMA,
        pltpu.SemaphoreType.DMA,
    ],
)
def _w2v_dot(a_st, b_st, out_hbm, a0, a1, b0, b1, out_v, sem0, sem1):
    wid = lax.axis_index("s") * NC + lax.axis_index("c")
    base = wid * BPW

    a_bufs = (a0, a1)
    b_bufs = (b0, b1)
    sems = (sem0, sem1)

    def start(c):
        slot = c % 2
        off = (base + c * CH2) * DIM
        ca = pltpu.async_copy(a_st.at[pl.ds(off, CB)], a_bufs[slot], sems[slot])
        cb = pltpu.async_copy(b_st.at[pl.ds(off, CB)], b_bufs[slot], sems[slot])
        return ca, cb

    pending = start(0)
    for c in range(NCH2):
        nxt = start(c + 1) if c + 1 < NCH2 else None
        for cp in pending:
            cp.wait()
        slot = c % 2
        _dot_chunk(a_bufs[slot], b_bufs[slot], out_v, c * CH2)
        pending = nxt

    pltpu.sync_copy(out_v, out_hbm.at[wid])


def kernel(center_words, context_words, input_embed, output_embed):
    a_st, b_st = _w2v_extract(
        center_words, context_words, input_embed.T, output_embed.T)
    out = _w2v_dot(a_st, b_st)
    return out.reshape(BATCH)


# speedup vs baseline: 2.9846x; 1.5155x over previous
import functools

import jax
import jax.numpy as jnp
from jax import lax
from jax.experimental import pallas as pl
from jax.experimental.pallas import tpu as pltpu
from jax.experimental.pallas import tpu_sc as plsc

VOCAB = 1_000_000
DIM = 64
BATCH = 16384

NC = 2
NS = 16
NW = NC * NS
BPW = BATCH // NW
L = 16

CW = 512
WIN = 62464
NCH = WIN // CW
XTRA_LO = 15 * WIN + NCH * CW
TAIL_LO = XTRA_LO + CW
TAIL_N = VOCAB - TAIL_LO
NIDXV = BATCH // L


def _popcount(mask):
    return plsc.all_reduce_population_count(mask)[0]


def _permute(x, idx):
    dnums = lax.GatherDimensionNumbers(
        offset_dims=(), collapsed_slice_dims=(0,), start_index_map=(0,)
    )
    return lax.gather(
        x, idx[:, None], dnums, (1,),
        mode=lax.GatherScatterMode.PROMISE_IN_BOUNDS,
    )


@functools.partial(
    pl.kernel,
    out_type=(
        jax.ShapeDtypeStruct((BATCH * DIM,), jnp.float32),
        jax.ShapeDtypeStruct((BATCH * DIM,), jnp.float32),
    ),
    mesh=plsc.VectorSubcoreMesh(core_axis_name="c", subcore_axis_name="s"),
    compiler_params=pltpu.CompilerParams(needs_layout_passes=False),
    scratch_types=[
        pltpu.VMEM((BATCH,), jnp.int32),
        pltpu.VMEM((BATCH + L,), jnp.int32),
        pltpu.VMEM((BATCH + L,), jnp.int32),
        pltpu.VMEM((DIM, CW), jnp.float32),
        pltpu.VMEM((DIM, CW), jnp.float32),
        pltpu.VMEM((DIM, TAIL_N), jnp.float32),
        pltpu.VMEM((L, DIM), jnp.float32),
        pltpu.SemaphoreType.DMA,
        pltpu.SemaphoreType.DMA,
        pltpu.SemaphoreType.DMA,
    ],
)
def _w2v_extract(idx_c_hbm, idx_x_hbm, ta, tb, out_a, out_b,
                 idxv, bkt_w, bkt_p, ch0, ch1, chtail,
                 rowbufs, sem0, sem1, wsem):
    core = lax.axis_index("c")
    tid = lax.axis_index("s")
    lanes = lax.iota(jnp.int32, L)
    bufs = (ch0, ch1)
    sems = (sem0, sem1)

    lo = tid * WIN
    hi = jnp.where(tid == NS - 1, VOCAB, lo + WIN)

    def pipe(tab, idx_hbm, out_hbm):
        pltpu.sync_copy(idx_hbm, idxv)

        def scan_body(v, cnt):
            w = idxv[pl.ds(v * L, L)]
            pos = v * L + lanes
            m = (w >= lo) & (w < hi)
            sk, sv, _ = plsc.sort_key_val(w, pos, mask=m)
            bkt_w[pl.ds(cnt, L)] = sk
            bkt_p[pl.ds(cnt, L)] = sv
            return cnt + _popcount(m)

        cnt = lax.fori_loop(0, NIDXV, scan_body, 0)
        nbv = (cnt + L - 1) >> 4

        def extract(chunk_lo, buf, cw):
            def bv_body(v, carry):
                w = bkt_w[pl.ds(v * L, L)]
                p = bkt_p[pl.ds(v * L, L)]
                m = (w >= chunk_lo) & (w < chunk_lo + cw)
                cm = _popcount(m)

                @pl.when(cm > 0)
                def _():
                    wv_abs, pv, _ = plsc.sort_key_val(w, p, mask=m)
                    wv = wv_abs - chunk_lo

                    def e_body(e, carry2):
                        ev = jnp.full((L,), e, jnp.int32)
                        col = _permute(wv, ev)
                        ps = _permute(pv, ev)[0]
                        for g in range(DIM // L):
                            vals = plsc.load_gather(buf, [lanes + g * L, col])
                            rowbufs[e, pl.ds(g * L, L)] = vals
                        off = pl.multiple_of(ps * DIM, 8)
                        pltpu.async_copy(
                            rowbufs.at[e], out_hbm.at[pl.ds(off, DIM)], wsem)
                        return carry2

                    lax.fori_loop(0, cm, e_body, 0)

                    def drain_body(e, carry2):
                        pltpu.make_async_copy(
                            rowbufs.at[0], out_hbm.at[pl.ds(0, DIM)], wsem
                        ).wait()
                        return carry2

                    lax.fori_loop(0, cm, drain_body, 0)

                return carry

            lax.fori_loop(0, nbv, bv_body, 0)

        def issue(c, slot):
            off = pl.multiple_of(lo + c * CW, 128)
            pltpu.async_copy(tab.at[:, pl.ds(off, CW)], bufs[slot], sems[slot])

        def wait(slot):
            pltpu.make_async_copy(
                tab.at[:, pl.ds(0, CW)], bufs[slot], sems[slot]).wait()

        issue(0, 0)
        issue(1, 1)

        def pair_body(c2, carry):
            c = 2 * c2
            wait(0)
            extract(lo + c * CW, bufs[0], CW)

            @pl.when(c + 2 < NCH)
            def _():
                issue(c + 2, 0)

            wait(1)
            extract(lo + (c + 1) * CW, bufs[1], CW)

            @pl.when(c + 3 < NCH)
            def _():
                issue(c + 3, 1)

            return carry

        lax.fori_loop(0, NCH // 2, pair_body, 0)

        @pl.when(tid == NS - 1)
        def _():
            pltpu.sync_copy(tab.at[:, pl.ds(XTRA_LO, CW)], ch0)
            extract(XTRA_LO, ch0, CW)
            pltpu.sync_copy(tab.at[:, pl.ds(TAIL_LO, TAIL_N)], chtail)
            extract(TAIL_LO, chtail, TAIL_N)

    @pl.when(core == 0)
    def _():
        pipe(ta, idx_c_hbm, out_a)

    @pl.when(core == 1)
    def _():
        pipe(tb, idx_x_hbm, out_b)


CH2 = 128
NCH2 = BPW // CH2
CB = CH2 * DIM


def _dot_chunk(a_ref, b_ref, out_ref, base):
    lanes = lax.iota(jnp.int32, L)
    rots = [(lanes + sh) % L for sh in (8, 4, 2, 1)]

    def group_body(g, carry):
        r0 = g * L
        acc = jnp.zeros((L,), jnp.float32)
        for k in range(L):
            f = (r0 + k) * DIM
            s = (
                a_ref[pl.ds(f, L)] * b_ref[pl.ds(f, L)]
                + a_ref[pl.ds(f + L, L)] * b_ref[pl.ds(f + L, L)]
                + a_ref[pl.ds(f + 2 * L, L)] * b_ref[pl.ds(f + 2 * L, L)]
                + a_ref[pl.ds(f + 3 * L, L)] * b_ref[pl.ds(f + 3 * L, L)]
            )
            for rot in rots:
                s = s + _permute(s, rot)
            acc = jnp.where(lanes == k, s, acc)
        out_ref[pl.ds(base + r0, L)] = acc
        return carry

    lax.fori_loop(0, CH2 // L, group_body, 0)


@functools.partial(
    pl.kernel,
    out_type=jax.ShapeDtypeStruct((NW, BPW), jnp.float32),
    mesh=plsc.VectorSubcoreMesh(core_axis_name="c", subcore_axis_name="s"),
    scratch_types=[
        pltpu.VMEM((CB,), jnp.float32),
        pltpu.VMEM((CB,), jnp.float32),
        pltpu.VMEM((CB,), jnp.float32),
        pltpu.VMEM((CB,), jnp.float32),
        pltpu.VMEM((BPW,), jnp.float32),
        pltpu.SemaphoreType.DMA,
        pltpu.SemaphoreType.DMA,
    ],
)
def _w2v_dot(a_st, b_st, out_hbm, a0, a1, b0, b1, out_v, sem0, sem1):
    wid = lax.axis_index("s") * NC + lax.axis_index("c")
    base = wid * BPW

    a_bufs = (a0, a1)
    b_bufs = (b0, b1)
    sems = (sem0, sem1)

    def start(c):
        slot = c % 2
        off = (base + c * CH2) * DIM
        ca = pltpu.async_copy(a_st.at[pl.ds(off, CB)], a_bufs[slot], sems[slot])
        cb = pltpu.async_copy(b_st.at[pl.ds(off, CB)], b_bufs[slot], sems[slot])
        return ca, cb

    pending = start(0)
    for c in range(NCH2):
        nxt = start(c + 1) if c + 1 < NCH2 else None
        for cp in pending:
            cp.wait()
        slot = c % 2
        _dot_chunk(a_bufs[slot], b_bufs[slot], out_v, c * CH2)
        pending = nxt

    pltpu.sync_copy(out_v, out_hbm.at[wid])


def kernel(center_words, context_words, input_embed, output_embed):
    a_st, b_st = _w2v_extract(
        center_words, context_words, input_embed.T, output_embed.T)
    out = _w2v_dot(a_st, b_st)
    return out.reshape(BATCH)


# speedup vs baseline: 3.8918x; 1.3040x over previous
import functools

import jax
import jax.numpy as jnp
from jax import lax
from jax.experimental import pallas as pl
from jax.experimental.pallas import tpu as pltpu
from jax.experimental.pallas import tpu_sc as plsc

VOCAB = 1_000_000
DIM = 64
BATCH = 16384

NC = 2
NS = 16
NW = NC * NS
BPW = BATCH // NW
L = 16

CW = 512
WIN = 62464
NCH = WIN // CW
XTRA_LO = 15 * WIN + NCH * CW
TAIL_LO = XTRA_LO + CW
TAIL_N = VOCAB - TAIL_LO
IDXB = 4096
SUBW = 8192
SUBSH = 13
NSB = 8
POSB = 14


def _popcount(mask):
    return plsc.all_reduce_population_count(mask)[0]


def _permute(x, idx):
    dnums = lax.GatherDimensionNumbers(
        offset_dims=(), collapsed_slice_dims=(0,), start_index_map=(0,)
    )
    return lax.gather(
        x, idx[:, None], dnums, (1,),
        mode=lax.GatherScatterMode.PROMISE_IN_BOUNDS,
    )


@functools.partial(
    pl.kernel,
    out_type=(
        jax.ShapeDtypeStruct((BATCH * DIM,), jnp.float32),
        jax.ShapeDtypeStruct((BATCH * DIM,), jnp.float32),
    ),
    mesh=plsc.VectorSubcoreMesh(core_axis_name="c", subcore_axis_name="s"),
    compiler_params=pltpu.CompilerParams(needs_layout_passes=False),
    scratch_types=[
        pltpu.VMEM((IDXB,), jnp.int32),
        pltpu.VMEM((BATCH + L,), jnp.int32),
        pltpu.VMEM((BATCH + L,), jnp.int32),
        pltpu.VMEM((BATCH + L,), jnp.int32),
        pltpu.VMEM((DIM, CW), jnp.float32),
        pltpu.VMEM((DIM, CW), jnp.float32),
        pltpu.VMEM((DIM, TAIL_N), jnp.float32),
        pltpu.VMEM((L, DIM), jnp.float32),
        pltpu.SemaphoreType.DMA,
        pltpu.SemaphoreType.DMA,
        pltpu.SemaphoreType.DMA,
    ],
)
def _w2v_extract(idx_c_hbm, idx_x_hbm, ta, tb, out_a, out_b,
                 idxb, bkt_w, bkt_p, sb, ch0, ch1, chtail,
                 rowbufs, sem0, sem1, wsem):
    core = lax.axis_index("c")
    tid = lax.axis_index("s")
    lanes = lax.iota(jnp.int32, L)
    bufs = (ch0, ch1)
    sems = (sem0, sem1)

    lo = tid * WIN
    hi = jnp.where(tid == NS - 1, VOCAB, lo + WIN)

    def pipe(tab, idx_hbm, out_hbm):
        cnt = 0
        for s4 in range(BATCH // IDXB):
            pltpu.sync_copy(idx_hbm.at[pl.ds(s4 * IDXB, IDXB)], idxb)
            pbase = s4 * IDXB

            def scan_body(v, c, pbase=pbase):
                w = idxb[pl.ds(v * L, L)]
                pos = pbase + v * L + lanes
                m = (w >= lo) & (w < hi)
                sk, sv, _ = plsc.sort_key_val(w, pos, mask=m)
                bkt_w[pl.ds(c, L)] = sk
                bkt_p[pl.ds(c, L)] = sv
                return c + _popcount(m)

            cnt = lax.fori_loop(0, IDXB // L, scan_body, cnt)
        nbv = (cnt + L - 1) >> 4

        offs_vec = jnp.zeros((L,), jnp.int32)
        ends_vec = jnp.zeros((L,), jnp.int32)
        cnt2 = 0
        for b in range(NSB):
            sub_lo = lo + b * SUBW
            sub_hi = hi if b == NSB - 1 else sub_lo + SUBW
            offs_vec = jnp.where(lanes == b, cnt2, offs_vec)

            def split_body(v, c, sub_lo=sub_lo, sub_hi=sub_hi):
                w = bkt_w[pl.ds(v * L, L)]
                p = bkt_p[pl.ds(v * L, L)]
                m = (w >= sub_lo) & (w < sub_hi)
                wl = jnp.clip(w - sub_lo, -1, SUBW)
                packed = (wl << POSB) | p
                sk, _, _ = plsc.sort_key_val(packed, p, mask=m)
                sb[pl.ds(c, L)] = sk
                return c + _popcount(m)

            cnt2 = lax.fori_loop(0, nbv, split_body, cnt2)
            ends_vec = jnp.where(lanes == b, cnt2, ends_vec)

        def extract(chunk_lo, buf, cw):
            sub = (chunk_lo - lo) >> SUBSH
            sub_lo = lo + (sub << SUBSH)
            clo = chunk_lo - sub_lo
            p_lo = clo << POSB
            p_hi = (clo + cw) << POSB
            subv = jnp.full((L,), sub, jnp.int32)
            start = _permute(offs_vec, subv)[0]
            end = _permute(ends_vec, subv)[0]

            def bv_body(v, carry):
                gi = v * L + lanes
                e = sb[pl.ds(v * L, L)]
                m = (e >= p_lo) & (e < p_hi) & (gi >= start) & (gi < end)
                cm = _popcount(m)

                @pl.when(cm > 0)
                def _():
                    sk, _, _ = plsc.sort_key_val(e, e, mask=m)

                    def e_body(ei, carry2):
                        ev = jnp.full((L,), ei, jnp.int32)
                        pk = _permute(sk, ev)
                        col = (pk >> POSB) - clo
                        ps = (pk & (2 ** POSB - 1))[0]
                        for g in range(DIM // L):
                            vals = plsc.load_gather(buf, [lanes + g * L, col])
                            rowbufs[ei, pl.ds(g * L, L)] = vals
                        off = pl.multiple_of(ps * DIM, 8)
                        pltpu.async_copy(
                            rowbufs.at[ei], out_hbm.at[pl.ds(off, DIM)], wsem)
                        return carry2

                    lax.fori_loop(0, cm, e_body, 0)

                    def drain_body(ei, carry2):
                        pltpu.make_async_copy(
                            rowbufs.at[0], out_hbm.at[pl.ds(0, DIM)], wsem
                        ).wait()
                        return carry2

                    lax.fori_loop(0, cm, drain_body, 0)

                return carry

            lax.fori_loop(start >> 4, (end + L - 1) >> 4, bv_body, 0)

        def issue(c, slot):
            off = pl.multiple_of(lo + c * CW, 128)
            pltpu.async_copy(tab.at[:, pl.ds(off, CW)], bufs[slot], sems[slot])

        def wait(slot):
            pltpu.make_async_copy(
                tab.at[:, pl.ds(0, CW)], bufs[slot], sems[slot]).wait()

        issue(0, 0)
        issue(1, 1)

        def pair_body(c2, carry):
            c = 2 * c2
            wait(0)
            extract(lo + c * CW, bufs[0], CW)

            @pl.when(c + 2 < NCH)
            def _():
                issue(c + 2, 0)

            wait(1)
            extract(lo + (c + 1) * CW, bufs[1], CW)

            @pl.when(c + 3 < NCH)
            def _():
                issue(c + 3, 1)

            return carry

        lax.fori_loop(0, NCH // 2, pair_body, 0)

        @pl.when(tid == NS - 1)
        def _():
            pltpu.sync_copy(tab.at[:, pl.ds(XTRA_LO, CW)], ch0)
            extract(XTRA_LO, ch0, CW)
            pltpu.sync_copy(tab.at[:, pl.ds(TAIL_LO, TAIL_N)], chtail)
            extract(TAIL_LO, chtail, TAIL_N)

    @pl.when(core == 0)
    def _():
        pipe(ta, idx_c_hbm, out_a)

    @pl.when(core == 1)
    def _():
        pipe(tb, idx_x_hbm, out_b)


CH2 = 128
NCH2 = BPW // CH2
CB = CH2 * DIM


def _dot_chunk(a_ref, b_ref, out_ref, base):
    lanes = lax.iota(jnp.int32, L)
    rots = [(lanes + sh) % L for sh in (8, 4, 2, 1)]

    def group_body(g, carry):
        r0 = g * L
        acc = jnp.zeros((L,), jnp.float32)
        for k in range(L):
            f = (r0 + k) * DIM
            s = (
                a_ref[pl.ds(f, L)] * b_ref[pl.ds(f, L)]
                + a_ref[pl.ds(f + L, L)] * b_ref[pl.ds(f + L, L)]
                + a_ref[pl.ds(f + 2 * L, L)] * b_ref[pl.ds(f + 2 * L, L)]
                + a_ref[pl.ds(f + 3 * L, L)] * b_ref[pl.ds(f + 3 * L, L)]
            )
            for rot in rots:
                s = s + _permute(s, rot)
            acc = jnp.where(lanes == k, s, acc)
        out_ref[pl.ds(base + r0, L)] = acc
        return carry

    lax.fori_loop(0, CH2 // L, group_body, 0)


@functools.partial(
    pl.kernel,
    out_type=jax.ShapeDtypeStruct((NW, BPW), jnp.float32),
    mesh=plsc.VectorSubcoreMesh(core_axis_name="c", subcore_axis_name="s"),
    scratch_types=[
        pltpu.VMEM((CB,), jnp.float32),
        pltpu.VMEM((CB,), jnp.float32),
        pltpu.VMEM((CB,), jnp.float32),
        pltpu.VMEM((CB,), jnp.float32),
        pltpu.VMEM((BPW,), jnp.float32),
        pltpu.SemaphoreType.DMA,
        pltpu.SemaphoreType.DMA,
    ],
)
def _w2v_dot(a_st, b_st, out_hbm, a0, a1, b0, b1, out_v, sem0, sem1):
    wid = lax.axis_index("s") * NC + lax.axis_index("c")
    base = wid * BPW

    a_bufs = (a0, a1)
    b_bufs = (b0, b1)
    sems = (sem0, sem1)

    def start(c):
        slot = c % 2
        off = (base + c * CH2) * DIM
        ca = pltpu.async_copy(a_st.at[pl.ds(off, CB)], a_bufs[slot], sems[slot])
        cb = pltpu.async_copy(b_st.at[pl.ds(off, CB)], b_bufs[slot], sems[slot])
        return ca, cb

    pending = start(0)
    for c in range(NCH2):
        nxt = start(c + 1) if c + 1 < NCH2 else None
        for cp in pending:
            cp.wait()
        slot = c % 2
        _dot_chunk(a_bufs[slot], b_bufs[slot], out_v, c * CH2)
        pending = nxt

    pltpu.sync_copy(out_v, out_hbm.at[wid])


def kernel(center_words, context_words, input_embed, output_embed):
    a_st, b_st = _w2v_extract(
        center_words, context_words, input_embed.T, output_embed.T)
    out = _w2v_dot(a_st, b_st)
    return out.reshape(BATCH)


# speedup vs baseline: 3.9591x; 1.0173x over previous
import functools

import jax
import jax.numpy as jnp
from jax import lax
from jax.experimental import pallas as pl
from jax.experimental.pallas import tpu as pltpu
from jax.experimental.pallas import tpu_sc as plsc

VOCAB = 1_000_000
DIM = 64
BATCH = 16384

NC = 2
NS = 16
NW = NC * NS
BPW = BATCH // NW
L = 16

CW = 512
WIN = 62464
NCH = WIN // CW
XTRA_LO = 15 * WIN + NCH * CW
TAIL_LO = XTRA_LO + CW
TAIL_N = VOCAB - TAIL_LO
IDXB = 4096
SUBW = 8192
SUBSH = 13
NSB = 8
POSB = 14


def _popcount(mask):
    return plsc.all_reduce_population_count(mask)[0]


def _permute(x, idx):
    dnums = lax.GatherDimensionNumbers(
        offset_dims=(), collapsed_slice_dims=(0,), start_index_map=(0,)
    )
    return lax.gather(
        x, idx[:, None], dnums, (1,),
        mode=lax.GatherScatterMode.PROMISE_IN_BOUNDS,
    )


@functools.partial(
    pl.kernel,
    out_type=(
        jax.ShapeDtypeStruct((BATCH * DIM,), jnp.float32),
        jax.ShapeDtypeStruct((BATCH * DIM,), jnp.float32),
    ),
    mesh=plsc.VectorSubcoreMesh(core_axis_name="c", subcore_axis_name="s"),
    compiler_params=pltpu.CompilerParams(needs_layout_passes=False),
    scratch_types=[
        pltpu.VMEM((IDXB,), jnp.int32),
        pltpu.VMEM((BATCH + L,), jnp.int32),
        pltpu.VMEM((BATCH + L,), jnp.int32),
        pltpu.VMEM((BATCH + L,), jnp.int32),
        pltpu.VMEM((DIM, CW), jnp.float32),
        pltpu.VMEM((DIM, CW), jnp.float32),
        pltpu.VMEM((DIM, TAIL_N), jnp.float32),
        pltpu.VMEM((L, DIM), jnp.float32),
        pltpu.SemaphoreType.DMA,
        pltpu.SemaphoreType.DMA,
        pltpu.SemaphoreType.DMA,
    ],
)
def _w2v_extract(idx_c_hbm, idx_x_hbm, ta, tb, out_a, out_b,
                 idxb, bkt_w, bkt_p, sb, ch0, ch1, chtail,
                 rowbufs, sem0, sem1, wsem):
    core = lax.axis_index("c")
    tid = lax.axis_index("s")
    lanes = lax.iota(jnp.int32, L)
    bufs = (ch0, ch1)
    sems = (sem0, sem1)

    lo = tid * WIN
    hi = jnp.where(tid == NS - 1, VOCAB, lo + WIN)

    def pipe(tab, idx_hbm, out_hbm):
        def issue(c, slot):
            off = pl.multiple_of(lo + c * CW, 128)
            pltpu.async_copy(tab.at[:, pl.ds(off, CW)], bufs[slot], sems[slot])

        issue(0, 0)
        issue(1, 1)

        cnt = 0
        for s4 in range(BATCH // IDXB):
            pltpu.sync_copy(idx_hbm.at[pl.ds(s4 * IDXB, IDXB)], idxb)
            pbase = s4 * IDXB

            def scan_body(v, c, pbase=pbase):
                w = idxb[pl.ds(v * L, L)]
                pos = pbase + v * L + lanes
                m = (w >= lo) & (w < hi)
                sk, sv, _ = plsc.sort_key_val(w, pos, mask=m)
                bkt_w[pl.ds(c, L)] = sk
                bkt_p[pl.ds(c, L)] = sv
                return c + _popcount(m)

            cnt = lax.fori_loop(0, IDXB // L, scan_body, cnt)
        nbv = (cnt + L - 1) >> 4

        offs_vec = jnp.zeros((L,), jnp.int32)
        ends_vec = jnp.zeros((L,), jnp.int32)
        cnt2 = 0
        for b in range(NSB):
            sub_lo = lo + b * SUBW
            sub_hi = hi if b == NSB - 1 else sub_lo + SUBW
            offs_vec = jnp.where(lanes == b, cnt2, offs_vec)

            def split_body(v, c, sub_lo=sub_lo, sub_hi=sub_hi):
                w = bkt_w[pl.ds(v * L, L)]
                p = bkt_p[pl.ds(v * L, L)]
                m = (w >= sub_lo) & (w < sub_hi)
                wl = jnp.clip(w - sub_lo, -1, SUBW)
                packed = (wl << POSB) | p
                sk, _, _ = plsc.sort_key_val(packed, p, mask=m)
                sb[pl.ds(c, L)] = sk
                return c + _popcount(m)

            cnt2 = lax.fori_loop(0, nbv, split_body, cnt2)
            ends_vec = jnp.where(lanes == b, cnt2, ends_vec)

        def extract(chunk_lo, buf, cw):
            sub = (chunk_lo - lo) >> SUBSH
            sub_lo = lo + (sub << SUBSH)
            clo = chunk_lo - sub_lo
            p_lo = clo << POSB
            p_hi = (clo + cw) << POSB
            subv = jnp.full((L,), sub, jnp.int32)
            start = _permute(offs_vec, subv)[0]
            end = _permute(ends_vec, subv)[0]

            def bv_body(v, carry):
                gi = v * L + lanes
                e = sb[pl.ds(v * L, L)]
                m = (e >= p_lo) & (e < p_hi) & (gi >= start) & (gi < end)
                cm = _popcount(m)

                @pl.when(cm > 0)
                def _():
                    sk, _, _ = plsc.sort_key_val(e, e, mask=m)

                    def e_body(ei, carry2):
                        ev = jnp.full((L,), ei, jnp.int32)
                        pk = _permute(sk, ev)
                        col = (pk >> POSB) - clo
                        ps = (pk & (2 ** POSB - 1))[0]
                        for g in range(DIM // L):
                            vals = plsc.load_gather(buf, [lanes + g * L, col])
                            rowbufs[ei, pl.ds(g * L, L)] = vals
                        off = pl.multiple_of(ps * DIM, 8)
                        pltpu.async_copy(
                            rowbufs.at[ei], out_hbm.at[pl.ds(off, DIM)], wsem)
                        return carry2

                    lax.fori_loop(0, cm, e_body, 0)

                    def drain_body(ei, carry2):
                        pltpu.make_async_copy(
                            rowbufs.at[0], out_hbm.at[pl.ds(0, DIM)], wsem
                        ).wait()
                        return carry2

                    lax.fori_loop(0, cm, drain_body, 0)

                return carry

            lax.fori_loop(start >> 4, (end + L - 1) >> 4, bv_body, 0)

        def wait(slot):
            pltpu.make_async_copy(
                tab.at[:, pl.ds(0, CW)], bufs[slot], sems[slot]).wait()

        def pair_body(c2, carry):
            c = 2 * c2
            wait(0)
            extract(lo + c * CW, bufs[0], CW)

            @pl.when(c + 2 < NCH)
            def _():
                issue(c + 2, 0)

            wait(1)
            extract(lo + (c + 1) * CW, bufs[1], CW)

            @pl.when(c + 3 < NCH)
            def _():
                issue(c + 3, 1)

            return carry

        lax.fori_loop(0, NCH // 2, pair_body, 0)

        @pl.when(tid == NS - 1)
        def _():
            pltpu.sync_copy(tab.at[:, pl.ds(XTRA_LO, CW)], ch0)
            extract(XTRA_LO, ch0, CW)
            pltpu.sync_copy(tab.at[:, pl.ds(TAIL_LO, TAIL_N)], chtail)
            extract(TAIL_LO, chtail, TAIL_N)

    @pl.when(core == 0)
    def _():
        pipe(ta, idx_c_hbm, out_a)

    @pl.when(core == 1)
    def _():
        pipe(tb, idx_x_hbm, out_b)


CH2 = 128
NCH2 = BPW // CH2
CB = CH2 * DIM


def _dot_chunk(a_ref, b_ref, out_ref, base):
    lanes = lax.iota(jnp.int32, L)
    rots = [(lanes + sh) % L for sh in (8, 4, 2, 1)]

    def group_body(g, carry):
        r0 = g * L
        acc = jnp.zeros((L,), jnp.float32)
        for k in range(L):
            f = (r0 + k) * DIM
            s = (
                a_ref[pl.ds(f, L)] * b_ref[pl.ds(f, L)]
                + a_ref[pl.ds(f + L, L)] * b_ref[pl.ds(f + L, L)]
                + a_ref[pl.ds(f + 2 * L, L)] * b_ref[pl.ds(f + 2 * L, L)]
                + a_ref[pl.ds(f + 3 * L, L)] * b_ref[pl.ds(f + 3 * L, L)]
            )
            for rot in rots:
                s = s + _permute(s, rot)
            acc = jnp.where(lanes == k, s, acc)
        out_ref[pl.ds(base + r0, L)] = acc
        return carry

    lax.fori_loop(0, CH2 // L, group_body, 0)


@functools.partial(
    pl.kernel,
    out_type=jax.ShapeDtypeStruct((NW, BPW), jnp.float32),
    mesh=plsc.VectorSubcoreMesh(core_axis_name="c", subcore_axis_name="s"),
    scratch_types=[
        pltpu.VMEM((CB,), jnp.float32),
        pltpu.VMEM((CB,), jnp.float32),
        pltpu.VMEM((CB,), jnp.float32),
        pltpu.VMEM((CB,), jnp.float32),
        pltpu.VMEM((BPW,), jnp.float32),
        pltpu.SemaphoreType.DMA,
        pltpu.SemaphoreType.DMA,
    ],
)
def _w2v_dot(a_st, b_st, out_hbm, a0, a1, b0, b1, out_v, sem0, sem1):
    wid = lax.axis_index("s") * NC + lax.axis_index("c")
    base = wid * BPW

    a_bufs = (a0, a1)
    b_bufs = (b0, b1)
    sems = (sem0, sem1)

    def start(c):
        slot = c % 2
        off = (base + c * CH2) * DIM
        ca = pltpu.async_copy(a_st.at[pl.ds(off, CB)], a_bufs[slot], sems[slot])
        cb = pltpu.async_copy(b_st.at[pl.ds(off, CB)], b_bufs[slot], sems[slot])
        return ca, cb

    pending = start(0)
    for c in range(NCH2):
        nxt = start(c + 1) if c + 1 < NCH2 else None
        for cp in pending:
            cp.wait()
        slot = c % 2
        _dot_chunk(a_bufs[slot], b_bufs[slot], out_v, c * CH2)
        pending = nxt

    pltpu.sync_copy(out_v, out_hbm.at[wid])


def kernel(center_words, context_words, input_embed, output_embed):
    a_st, b_st = _w2v_extract(
        center_words, context_words, input_embed.T, output_embed.T)
    out = _w2v_dot(a_st, b_st)
    return out.reshape(BATCH)
